# Initial kernel scaffold; baseline (speedup 1.0000x reference)
import functools

import jax
import jax.numpy as jnp
from jax.experimental import pallas as pl

N_TOK = 16384
DIM = 2048
N_EXPERTS = 64
TOPK = 8
N_GROUPS = 8
GROUP_SIZE = N_EXPERTS // N_GROUPS
TOPK_GROUPS = 4
ROUTE_SCALE = 1.0

BN = 512

NEG_INF = jnp.float32(-jnp.inf)


def _gate_kernel(x_ref, w_ref, b_ref, wout_ref, iout_ref):
    logits = jax.lax.dot_general(
        w_ref[...], x_ref[...],
        dimension_numbers=(((1,), (1,)), ((), ())),
        preferred_element_type=jnp.float32,
    )
    scores = jax.nn.sigmoid(logits)
    s = scores + b_ref[...]

    bn = s.shape[1]
    s3 = s.reshape(N_GROUPS, GROUP_SIZE, bn)

    m1 = jnp.max(s3, axis=1, keepdims=True)
    sub_iota = jax.lax.broadcasted_iota(jnp.int32, s3.shape, 1)
    idx1 = jnp.min(jnp.where(s3 == m1, sub_iota, GROUP_SIZE),
                   axis=1, keepdims=True)
    m2 = jnp.max(jnp.where(sub_iota == idx1, NEG_INF, s3),
                 axis=1, keepdims=True)
    gscore = (m1 + m2)[:, 0, :]

    ga = gscore[:, None, :]
    gb = gscore[None, :, :]
    gidx = jax.lax.broadcasted_iota(jnp.int32, (N_GROUPS, N_GROUPS, bn), 0)
    kidx = jax.lax.broadcasted_iota(jnp.int32, (N_GROUPS, N_GROUPS, bn), 1)
    beats = (gb > ga) | ((gb == ga) & (kidx < gidx))
    rank = jnp.sum(beats.astype(jnp.int32), axis=1)
    keep = rank < TOPK_GROUPS

    masked = jnp.where(keep[:, None, :], s3, NEG_INF).reshape(N_EXPERTS, bn)

    eiota = jax.lax.broadcasted_iota(jnp.int32, (N_EXPERTS, bn), 0)
    wlist, ilist = [], []
    for _ in range(TOPK):
        m = jnp.max(masked, axis=0, keepdims=True)
        idx = jnp.min(jnp.where(masked == m, eiota, N_EXPERTS),
                      axis=0, keepdims=True)
        onehot = eiota == idx
        wk = jnp.sum(jnp.where(onehot, scores, 0.0), axis=0, keepdims=True)
        masked = jnp.where(onehot, NEG_INF, masked)
        wlist.append(wk)
        ilist.append(idx)

    w8 = jnp.concatenate(wlist, axis=0)
    i8 = jnp.concatenate(ilist, axis=0)
    wsum = jnp.sum(w8, axis=0, keepdims=True)
    wout_ref[...] = w8 / (wsum + 1e-6) * ROUTE_SCALE
    iout_ref[...] = i8


@functools.partial(jax.jit, static_argnames=())
def kernel(x, weight, bias):
    n = x.shape[0]
    grid = (n // BN,)
    wt, it = pl.pallas_call(
        _gate_kernel,
        grid=grid,
        in_specs=[
            pl.BlockSpec((BN, DIM), lambda i: (i, 0)),
            pl.BlockSpec((N_EXPERTS, DIM), lambda i: (0, 0)),
            pl.BlockSpec((N_EXPERTS, 1), lambda i: (0, 0)),
        ],
        out_specs=[
            pl.BlockSpec((TOPK, BN), lambda i: (0, i)),
            pl.BlockSpec((TOPK, BN), lambda i: (0, i)),
        ],
        out_shape=[
            jax.ShapeDtypeStruct((TOPK, n), jnp.float32),
            jax.ShapeDtypeStruct((TOPK, n), jnp.int32),
        ],
    )(x, weight, bias.reshape(N_EXPERTS, 1))
    return wt.T.astype(x.dtype), it.T


# speedup vs baseline: 7.7144x; 7.7144x over previous
import functools

import jax
import jax.numpy as jnp
from jax.experimental import pallas as pl

N_TOK = 16384
DIM = 2048
N_EXPERTS = 64
TOPK = 8
N_GROUPS = 8
GROUP_SIZE = N_EXPERTS // N_GROUPS
TOPK_GROUPS = 4
ROUTE_SCALE = 1.0

BN = 512

NEG_INF = float("-inf")


def _gate_kernel(x_ref, w_ref, b_ref, wout_ref, iout_ref):
    logits = jax.lax.dot_general(
        w_ref[...], x_ref[...],
        dimension_numbers=(((1,), (1,)), ((), ())),
        preferred_element_type=jnp.float32,
    )
    scores = jax.nn.sigmoid(logits)
    s = scores + b_ref[...]

    bn = s.shape[1]
    s3 = s.reshape(N_GROUPS, GROUP_SIZE, bn)

    m1 = jnp.max(s3, axis=1, keepdims=True)
    sub_iota = jax.lax.broadcasted_iota(jnp.int32, s3.shape, 1)
    idx1 = jnp.min(jnp.where(s3 == m1, sub_iota, GROUP_SIZE),
                   axis=1, keepdims=True)
    m2 = jnp.max(jnp.where(sub_iota == idx1, NEG_INF, s3),
                 axis=1, keepdims=True)
    gscore = (m1 + m2)[:, 0, :]

    ga = gscore[:, None, :]
    gb = gscore[None, :, :]
    gidx = jax.lax.broadcasted_iota(jnp.int32, (N_GROUPS, N_GROUPS, bn), 0)
    kidx = jax.lax.broadcasted_iota(jnp.int32, (N_GROUPS, N_GROUPS, bn), 1)
    beats = (gb > ga) | ((gb == ga) & (kidx < gidx))
    rank = jnp.sum(beats.astype(jnp.int32), axis=1)
    keep = rank < TOPK_GROUPS

    masked = jnp.where(keep[:, None, :], s3, NEG_INF).reshape(N_EXPERTS, bn)

    eiota = jax.lax.broadcasted_iota(jnp.int32, (N_EXPERTS, bn), 0)
    wlist, ilist = [], []
    for _ in range(TOPK):
        m = jnp.max(masked, axis=0, keepdims=True)
        idx = jnp.min(jnp.where(masked == m, eiota, N_EXPERTS),
                      axis=0, keepdims=True)
        onehot = eiota == idx
        wk = jnp.sum(jnp.where(onehot, scores, 0.0), axis=0, keepdims=True)
        masked = jnp.where(onehot, NEG_INF, masked)
        wlist.append(wk)
        ilist.append(idx)

    w8 = jnp.concatenate(wlist, axis=0)
    i8 = jnp.concatenate(ilist, axis=0)
    wsum = jnp.sum(w8, axis=0, keepdims=True)
    wout_ref[...] = w8 / (wsum + 1e-6) * ROUTE_SCALE
    iout_ref[...] = i8


@functools.partial(jax.jit, static_argnames=())
def kernel(x, weight, bias):
    n = x.shape[0]
    grid = (n // BN,)
    wt, it = pl.pallas_call(
        _gate_kernel,
        grid=grid,
        in_specs=[
            pl.BlockSpec((BN, DIM), lambda i: (i, 0)),
            pl.BlockSpec((N_EXPERTS, DIM), lambda i: (0, 0)),
            pl.BlockSpec((N_EXPERTS, 1), lambda i: (0, 0)),
        ],
        out_specs=[
            pl.BlockSpec((TOPK, BN), lambda i: (0, i)),
            pl.BlockSpec((TOPK, BN), lambda i: (0, i)),
        ],
        out_shape=[
            jax.ShapeDtypeStruct((TOPK, n), jnp.float32),
            jax.ShapeDtypeStruct((TOPK, n), jnp.int32),
        ],
    )(x, weight, bias.reshape(N_EXPERTS, 1))
    return wt.T.astype(x.dtype), it.T
